# baseline (speedup 1.0000x reference)
import functools

import jax
import jax.numpy as jnp
from jax import lax
from jax.experimental import pallas as pl
from jax.experimental.pallas import tpu as pltpu
from jax.experimental.pallas import tpu_sc as plsc

N = 10000
NP = 10240
E = 320000
EP = 327680
IN_DIM = 128
HID = 128
H1 = 8
OUT = 64
W1T = 144
W2T = 80
C = 128
SENT = N + 16

NC, NS, L = 2, 16, 16
HPC = H1 // NC
ROWS_T1 = (EP // C) // NS
ROWS_T2 = (EP // C) // (NC * NS)
NB = 1024
NBLK = NP // NB


def _d1_body(x_ref, w_ref, as_ref, ad_ref, tbl_ref, adst_ref):
    h = jnp.dot(x_ref[...], w_ref[...], preferred_element_type=jnp.float32)
    asv = jnp.sum(h * as_ref[0], axis=1)
    adv = jnp.sum(h * ad_ref[0], axis=1)
    tbl_ref[0, :, 0:IN_DIM] = h
    tbl_ref[0, :, IN_DIM:IN_DIM + 1] = asv[:, None]
    tbl_ref[0, :, IN_DIM + 1:IN_DIM + 2] = jnp.ones((NB, 1), jnp.float32)
    tbl_ref[0, :, IN_DIM + 2:W1T] = jnp.zeros((NB, W1T - IN_DIM - 2), jnp.float32)
    adst_ref[0] = adv.reshape(NB // 128, 128)


def _dense1(xp, W1, as1, ad1):
    return pl.pallas_call(
        _d1_body,
        grid=(H1, NBLK),
        in_specs=[
            pl.BlockSpec((NB, IN_DIM), lambda k, i: (i, 0)),
            pl.BlockSpec((IN_DIM, HID), lambda k, i: (0, k)),
            pl.BlockSpec((1, 1, HID), lambda k, i: (k, 0, 0)),
            pl.BlockSpec((1, 1, HID), lambda k, i: (k, 0, 0)),
        ],
        out_specs=[
            pl.BlockSpec((1, NB, W1T), lambda k, i: (k, i, 0)),
            pl.BlockSpec((1, NB // 128, 128), lambda k, i: (k, i, 0)),
        ],
        out_shape=[
            jax.ShapeDtypeStruct((H1, NP, W1T), jnp.float32),
            jax.ShapeDtypeStruct((H1, NP // 128, 128), jnp.float32),
        ],
    )(xp, W1, as1.reshape(H1, 1, HID), ad1.reshape(H1, 1, HID))


C1 = 80
CPH = EP // C1 // NS
G1 = 32


def _ep1_body(tbl_hbm, adst_hbm, src_hbm, dst_hbm, zero_hbm, out_hbm,
              acc, src_v, dst_v, adst_v, row_a, row_b, ex_v,
              sem_a, sem_b):
    c = lax.axis_index("c")
    t = lax.axis_index("s")
    rows_per_tile = NP // NS

    def compute_chunk(row_v, j):
        for g in range(C1 // L):
            d16 = dst_v[j, pl.ds(g * L, L)]
            av = plsc.load_gather(adst_v, [d16])
            rows = lax.iota(jnp.int32, L) + g * L
            cols = jnp.full((L,), IN_DIM, jnp.int32)
            asv = plsc.load_gather(row_v, [rows, cols])
            al = asv + av
            al = jnp.where(al >= 0.0, al, al * jnp.float32(0.2))
            ex_v[pl.ds(g * L, L)] = jnp.exp(al)

        def scale_body(e, carry3):
            s = ex_v[pl.ds(e, L)][0]
            for r in range(W1T // L):
                row_v[e, pl.ds(r * L, L)] = row_v[e, pl.ds(r * L, L)] * s
            return carry3
        lax.fori_loop(0, C1, scale_body, 0, unroll=16)

    def head_body(kk, carry):
        k = c * HPC + kk
        pltpu.sync_copy(zero_hbm.at[pl.ds(t * rows_per_tile, rows_per_tile)],
                        acc.at[pl.ds(t * rows_per_tile, rows_per_tile)])
        pltpu.sync_copy(adst_hbm.at[k], adst_v)
        plsc.subcore_barrier()

        def group_body(grp, carry1):
            base = t * CPH + grp * G1
            pltpu.sync_copy(src_hbm.at[pl.ds(base, G1)], src_v)
            pltpu.sync_copy(dst_hbm.at[pl.ds(base, G1)], dst_v)
            pltpu.async_copy(tbl_hbm.at[k].at[src_v.at[0]], row_a, sem_a)

            def pair_body(jj, carry2):
                j0 = jj * 2
                j1 = j0 + 1
                pltpu.async_copy(tbl_hbm.at[k].at[src_v.at[j1]], row_b, sem_b)
                pltpu.make_async_copy(tbl_hbm.at[k].at[src_v.at[j0]],
                                      row_a, sem_a).wait()
                compute_chunk(row_a, j0)
                pltpu.sync_copy(row_a, acc.at[dst_v.at[j0]], add=True)

                @pl.when(jj < G1 // 2 - 1)
                def _():
                    pltpu.async_copy(tbl_hbm.at[k].at[src_v.at[j0 + 2]],
                                     row_a, sem_a)
                pltpu.make_async_copy(tbl_hbm.at[k].at[src_v.at[j1]],
                                      row_b, sem_b).wait()
                compute_chunk(row_b, j1)
                pltpu.sync_copy(row_b, acc.at[dst_v.at[j1]], add=True)
                return carry2
            lax.fori_loop(0, G1 // 2, pair_body, 0)
            return carry1
        lax.fori_loop(0, CPH // G1, group_body, 0)
        plsc.subcore_barrier()
        pltpu.sync_copy(acc.at[pl.ds(t * rows_per_tile, rows_per_tile)],
                        out_hbm.at[k].at[pl.ds(t * rows_per_tile, rows_per_tile)])
        plsc.subcore_barrier()
        return carry
    lax.fori_loop(0, HPC, head_body, 0)


_ep1 = functools.partial(
    pl.kernel,
    out_type=jax.ShapeDtypeStruct((H1, NP, W1T), jnp.float32),
    mesh=plsc.VectorSubcoreMesh(core_axis_name="c", subcore_axis_name="s"),
    compiler_params=pltpu.CompilerParams(needs_layout_passes=False, use_tc_tiling_on_sc=False),
    scratch_types=[
        pltpu.VMEM_SHARED((NP, W1T), jnp.float32),
        pltpu.VMEM((G1, C1), jnp.int32),
        pltpu.VMEM((G1, C1), jnp.int32),
        pltpu.VMEM((NP,), jnp.float32),
        pltpu.VMEM((C1, W1T), jnp.float32),
        pltpu.VMEM((C1, W1T), jnp.float32),
        pltpu.VMEM((C1 + L,), jnp.float32),
        pltpu.SemaphoreType.DMA,
        pltpu.SemaphoreType.DMA,
    ],
)(_ep1_body)


def _d2_body(acc_ref, b1_ref, w2_ref, as_ref, ad_ref, tbl_ref, adst_ref):
    gs = []
    for k in range(H1):
        a = acc_ref[k]
        num = a[:, 0:IN_DIM]
        den = a[:, IN_DIM + 1:IN_DIM + 2]
        gs.append(jnp.maximum(num / (den + 1e-16) + b1_ref[k][None, :], 0.0))
    g = jnp.concatenate(gs, axis=1)
    h2 = jnp.dot(g, w2_ref[...], preferred_element_type=jnp.float32)
    asv = jnp.sum(h2 * as_ref[...], axis=1)
    adv = jnp.sum(h2 * ad_ref[...], axis=1)
    tbl_ref[:, 0:OUT] = h2
    tbl_ref[:, OUT:OUT + 1] = asv[:, None]
    tbl_ref[:, OUT + 1:OUT + 2] = jnp.ones((NB, 1), jnp.float32)
    tbl_ref[:, OUT + 2:W2T] = jnp.zeros((NB, W2T - OUT - 2), jnp.float32)
    adst_ref[0] = adv.reshape(NB // 128, 128)


def _dense2(acc1, b1, W2, as2, ad2):
    return pl.pallas_call(
        _d2_body,
        grid=(NBLK,),
        in_specs=[
            pl.BlockSpec((H1, NB, W1T), lambda i: (0, i, 0)),
            pl.BlockSpec((H1, HID), lambda i: (0, 0)),
            pl.BlockSpec((H1 * HID, OUT), lambda i: (0, 0)),
            pl.BlockSpec((1, OUT), lambda i: (0, 0)),
            pl.BlockSpec((1, OUT), lambda i: (0, 0)),
        ],
        out_specs=[
            pl.BlockSpec((NB, W2T), lambda i: (i, 0)),
            pl.BlockSpec((1, NB // 128, 128), lambda i: (i, 0, 0)),
        ],
        out_shape=[
            jax.ShapeDtypeStruct((NP, W2T), jnp.float32),
            jax.ShapeDtypeStruct((NBLK, NB // 128, 128), jnp.float32),
        ],
    )(acc1, b1.reshape(H1, HID), W2, as2.reshape(1, OUT), ad2.reshape(1, OUT))


def _ep2_body(tbl_hbm, adst_hbm, src_hbm, dst_hbm, zero_hbm, out_hbm,
              acc, src_v, dst_v, adst_v, row_a, row_b, ex_v,
              sem_a, sem_b):
    c = lax.axis_index("c")
    t = lax.axis_index("s")
    rows_per_tile = NP // NS
    w = c * NS + t
    pltpu.sync_copy(src_hbm.at[pl.ds(w * ROWS_T2, ROWS_T2)], src_v)
    pltpu.sync_copy(dst_hbm.at[pl.ds(w * ROWS_T2, ROWS_T2)], dst_v)
    pltpu.sync_copy(zero_hbm.at[pl.ds(t * rows_per_tile, rows_per_tile)],
                    acc.at[pl.ds(t * rows_per_tile, rows_per_tile)])
    pltpu.sync_copy(adst_hbm.at[0], adst_v)
    plsc.subcore_barrier()

    def compute_chunk(row_v, j):
        for g in range(C // L):
            d16 = dst_v[j, pl.ds(g * L, L)]
            av = plsc.load_gather(adst_v, [d16])
            rows = lax.iota(jnp.int32, L) + g * L
            cols = jnp.full((L,), OUT, jnp.int32)
            asv = plsc.load_gather(row_v, [rows, cols])
            al = asv + av
            al = jnp.where(al >= 0.0, al, al * jnp.float32(0.2))
            ex_v[pl.ds(g * L, L)] = jnp.exp(al)

        def scale_body(e, carry3):
            s = ex_v[pl.ds(e, L)][0]
            for r in range(W2T // L):
                row_v[e, pl.ds(r * L, L)] = row_v[e, pl.ds(r * L, L)] * s
            return carry3
        lax.fori_loop(0, C, scale_body, 0, unroll=16)

    pltpu.async_copy(tbl_hbm.at[src_v.at[0]], row_a, sem_a)

    def pair_body(jj, carry2):
        j0 = jj * 2
        j1 = j0 + 1
        pltpu.async_copy(tbl_hbm.at[src_v.at[j1]], row_b, sem_b)
        pltpu.make_async_copy(tbl_hbm.at[src_v.at[j0]], row_a, sem_a).wait()
        compute_chunk(row_a, j0)
        pltpu.sync_copy(row_a, acc.at[dst_v.at[j0]], add=True)

        @pl.when(jj < ROWS_T2 // 2 - 1)
        def _():
            pltpu.async_copy(tbl_hbm.at[src_v.at[j0 + 2]], row_a, sem_a)
        pltpu.make_async_copy(tbl_hbm.at[src_v.at[j1]], row_b, sem_b).wait()
        compute_chunk(row_b, j1)
        pltpu.sync_copy(row_b, acc.at[dst_v.at[j1]], add=True)
        return carry2
    lax.fori_loop(0, ROWS_T2 // 2, pair_body, 0)
    plsc.subcore_barrier()
    pltpu.sync_copy(acc.at[pl.ds(t * rows_per_tile, rows_per_tile)],
                    out_hbm.at[c].at[pl.ds(t * rows_per_tile, rows_per_tile)])


_ep2 = functools.partial(
    pl.kernel,
    out_type=jax.ShapeDtypeStruct((NC, NP, W2T), jnp.float32),
    mesh=plsc.VectorSubcoreMesh(core_axis_name="c", subcore_axis_name="s"),
    compiler_params=pltpu.CompilerParams(needs_layout_passes=False, use_tc_tiling_on_sc=False),
    scratch_types=[
        pltpu.VMEM_SHARED((NP, W2T), jnp.float32),
        pltpu.VMEM((ROWS_T2, C), jnp.int32),
        pltpu.VMEM((ROWS_T2, C), jnp.int32),
        pltpu.VMEM((NP,), jnp.float32),
        pltpu.VMEM((C, W2T), jnp.float32),
        pltpu.VMEM((C, W2T), jnp.float32),
        pltpu.VMEM((C + L,), jnp.float32),
        pltpu.SemaphoreType.DMA,
        pltpu.SemaphoreType.DMA,
    ],
)(_ep2_body)


def _f_body(acc_ref, b2_ref, out_ref):
    a = acc_ref[0] + acc_ref[1]
    z = a[:, 0:OUT] / (a[:, OUT + 1:OUT + 2] + 1e-16) + b2_ref[...]
    m = jnp.max(z, axis=1, keepdims=True)
    zz = z - m
    out_ref[...] = zz - jnp.log(jnp.sum(jnp.exp(zz), axis=1, keepdims=True))


def _final(acc2, b2):
    return pl.pallas_call(
        _f_body,
        grid=(NBLK,),
        in_specs=[
            pl.BlockSpec((NC, NB, W2T), lambda i: (0, i, 0)),
            pl.BlockSpec((1, OUT), lambda i: (0, 0)),
        ],
        out_specs=pl.BlockSpec((NB, OUT), lambda i: (i, 0)),
        out_shape=jax.ShapeDtypeStruct((NP, OUT), jnp.float32),
    )(acc2, b2.reshape(1, OUT))


@jax.jit
def kernel(x, edge_index, W1, att_src1, att_dst1, b1, W2, att_src2,
           att_dst2, b2):
    src = edge_index[0].astype(jnp.int32)
    dst = edge_index[1].astype(jnp.int32)
    pad = jnp.full((EP - E,), SENT, jnp.int32)
    srcf = jnp.concatenate([src, pad])
    dstf = jnp.concatenate([dst, pad])
    srcp1 = srcf.reshape(EP // C1, C1)
    dstp1 = dstf.reshape(EP // C1, C1)
    srcp = srcf.reshape(EP // C, C)
    dstp = dstf.reshape(EP // C, C)
    xp = jnp.zeros((NP, IN_DIM), jnp.float32).at[:N].set(x)
    z1 = jnp.zeros((NP, W1T), jnp.float32)
    z2 = jnp.zeros((NP, W2T), jnp.float32)

    tbl1, adst1 = _dense1(xp, W1, att_src1, att_dst1)
    acc1 = _ep1(tbl1, adst1.reshape(H1, NP), srcp1, dstp1, z1)
    tbl2, adst2 = _dense2(acc1, b1, W2, att_src2, att_dst2)
    acc2 = _ep2(tbl2, adst2.reshape(1, NP), srcp, dstp, z2)
    outp = _final(acc2, b2)
    return outp[:N]


# speedup vs baseline: 1.0014x; 1.0014x over previous
import functools

import jax
import jax.numpy as jnp
from jax import lax
from jax.experimental import pallas as pl
from jax.experimental.pallas import tpu as pltpu
from jax.experimental.pallas import tpu_sc as plsc

N = 10000
NP = 10240
E = 320000
EP = 327680
IN_DIM = 128
HID = 128
H1 = 8
OUT = 64
W1T = 144
W2T = 80
C = 128
SENT = N + 16

NC, NS, L = 2, 16, 16
HPC = H1 // NC
ROWS_T1 = (EP // C) // NS
ROWS_T2 = (EP // C) // (NC * NS)
NB = 1024
NBLK = NP // NB


def _d1_body(x_ref, w_ref, as_ref, ad_ref, tbl_ref, adst_ref):
    h = jnp.dot(x_ref[...], w_ref[...], preferred_element_type=jnp.float32)
    asv = jnp.sum(h * as_ref[0], axis=1)
    adv = jnp.sum(h * ad_ref[0], axis=1)
    tbl_ref[0, :, 0:IN_DIM] = h
    tbl_ref[0, :, IN_DIM:IN_DIM + 1] = asv[:, None]
    tbl_ref[0, :, IN_DIM + 1:IN_DIM + 2] = jnp.ones((NB, 1), jnp.float32)
    tbl_ref[0, :, IN_DIM + 2:W1T] = jnp.zeros((NB, W1T - IN_DIM - 2), jnp.float32)
    adst_ref[0] = adv.reshape(NB // 128, 128)


def _dense1(xp, W1, as1, ad1):
    return pl.pallas_call(
        _d1_body,
        grid=(H1, NBLK),
        in_specs=[
            pl.BlockSpec((NB, IN_DIM), lambda k, i: (i, 0)),
            pl.BlockSpec((IN_DIM, HID), lambda k, i: (0, k)),
            pl.BlockSpec((1, 1, HID), lambda k, i: (k, 0, 0)),
            pl.BlockSpec((1, 1, HID), lambda k, i: (k, 0, 0)),
        ],
        out_specs=[
            pl.BlockSpec((1, NB, W1T), lambda k, i: (k, i, 0)),
            pl.BlockSpec((1, NB // 128, 128), lambda k, i: (k, i, 0)),
        ],
        out_shape=[
            jax.ShapeDtypeStruct((H1, NP, W1T), jnp.float32),
            jax.ShapeDtypeStruct((H1, NP // 128, 128), jnp.float32),
        ],
    )(xp, W1, as1.reshape(H1, 1, HID), ad1.reshape(H1, 1, HID))


C1 = 64
CPH = EP // C1 // NS
G1 = 64


def _ep1_body(tbl_hbm, adst_hbm, src_hbm, dst_hbm, zero_hbm, out_hbm,
              acc, src_v, dst_v, adst_v, row_a, row_b, ex_v,
              sem_a, sem_b):
    c = lax.axis_index("c")
    t = lax.axis_index("s")
    rows_per_tile = NP // NS

    def compute_chunk(row_v, j):
        for g in range(C1 // L):
            d16 = dst_v[j, pl.ds(g * L, L)]
            av = plsc.load_gather(adst_v, [d16])
            rows = lax.iota(jnp.int32, L) + g * L
            cols = jnp.full((L,), IN_DIM, jnp.int32)
            asv = plsc.load_gather(row_v, [rows, cols])
            al = asv + av
            al = jnp.where(al >= 0.0, al, al * jnp.float32(0.2))
            ex_v[pl.ds(g * L, L)] = jnp.exp(al)

        def scale_body(e, carry3):
            s = ex_v[pl.ds(e, L)][0]
            for r in range(W1T // L):
                row_v[e, pl.ds(r * L, L)] = row_v[e, pl.ds(r * L, L)] * s
            return carry3
        lax.fori_loop(0, C1, scale_body, 0, unroll=16)

    def head_body(kk, carry):
        k = c * HPC + kk
        pltpu.sync_copy(zero_hbm.at[pl.ds(t * rows_per_tile, rows_per_tile)],
                        acc.at[pl.ds(t * rows_per_tile, rows_per_tile)])
        pltpu.sync_copy(adst_hbm.at[k], adst_v)
        plsc.subcore_barrier()

        def group_body(grp, carry1):
            base = t * CPH + grp * G1
            pltpu.sync_copy(src_hbm.at[pl.ds(base, G1)], src_v)
            pltpu.sync_copy(dst_hbm.at[pl.ds(base, G1)], dst_v)
            pltpu.async_copy(tbl_hbm.at[k].at[src_v.at[0]], row_a, sem_a)

            def pair_body(jj, carry2):
                j0 = jj * 2
                j1 = j0 + 1
                pltpu.async_copy(tbl_hbm.at[k].at[src_v.at[j1]], row_b, sem_b)
                pltpu.make_async_copy(tbl_hbm.at[k].at[src_v.at[j0]],
                                      row_a, sem_a).wait()
                compute_chunk(row_a, j0)
                pltpu.sync_copy(row_a, acc.at[dst_v.at[j0]], add=True)

                @pl.when(jj < G1 // 2 - 1)
                def _():
                    pltpu.async_copy(tbl_hbm.at[k].at[src_v.at[j0 + 2]],
                                     row_a, sem_a)
                pltpu.make_async_copy(tbl_hbm.at[k].at[src_v.at[j1]],
                                      row_b, sem_b).wait()
                compute_chunk(row_b, j1)
                pltpu.sync_copy(row_b, acc.at[dst_v.at[j1]], add=True)
                return carry2
            lax.fori_loop(0, G1 // 2, pair_body, 0)
            return carry1
        lax.fori_loop(0, CPH // G1, group_body, 0)
        plsc.subcore_barrier()
        pltpu.sync_copy(acc.at[pl.ds(t * rows_per_tile, rows_per_tile)],
                        out_hbm.at[k].at[pl.ds(t * rows_per_tile, rows_per_tile)])
        plsc.subcore_barrier()
        return carry
    lax.fori_loop(0, HPC, head_body, 0)


_ep1 = functools.partial(
    pl.kernel,
    out_type=jax.ShapeDtypeStruct((H1, NP, W1T), jnp.float32),
    mesh=plsc.VectorSubcoreMesh(core_axis_name="c", subcore_axis_name="s"),
    compiler_params=pltpu.CompilerParams(needs_layout_passes=False, use_tc_tiling_on_sc=False),
    scratch_types=[
        pltpu.VMEM_SHARED((NP, W1T), jnp.float32),
        pltpu.VMEM((G1, C1), jnp.int32),
        pltpu.VMEM((G1, C1), jnp.int32),
        pltpu.VMEM((NP,), jnp.float32),
        pltpu.VMEM((C1, W1T), jnp.float32),
        pltpu.VMEM((C1, W1T), jnp.float32),
        pltpu.VMEM((C1 + L,), jnp.float32),
        pltpu.SemaphoreType.DMA,
        pltpu.SemaphoreType.DMA,
    ],
)(_ep1_body)


def _d2_body(acc_ref, b1_ref, w2_ref, as_ref, ad_ref, tbl_ref, adst_ref):
    gs = []
    for k in range(H1):
        a = acc_ref[k]
        num = a[:, 0:IN_DIM]
        den = a[:, IN_DIM + 1:IN_DIM + 2]
        gs.append(jnp.maximum(num / (den + 1e-16) + b1_ref[k][None, :], 0.0))
    g = jnp.concatenate(gs, axis=1)
    h2 = jnp.dot(g, w2_ref[...], preferred_element_type=jnp.float32)
    asv = jnp.sum(h2 * as_ref[...], axis=1)
    adv = jnp.sum(h2 * ad_ref[...], axis=1)
    tbl_ref[:, 0:OUT] = h2
    tbl_ref[:, OUT:OUT + 1] = asv[:, None]
    tbl_ref[:, OUT + 1:OUT + 2] = jnp.ones((NB, 1), jnp.float32)
    tbl_ref[:, OUT + 2:W2T] = jnp.zeros((NB, W2T - OUT - 2), jnp.float32)
    adst_ref[0] = adv.reshape(NB // 128, 128)


def _dense2(acc1, b1, W2, as2, ad2):
    return pl.pallas_call(
        _d2_body,
        grid=(NBLK,),
        in_specs=[
            pl.BlockSpec((H1, NB, W1T), lambda i: (0, i, 0)),
            pl.BlockSpec((H1, HID), lambda i: (0, 0)),
            pl.BlockSpec((H1 * HID, OUT), lambda i: (0, 0)),
            pl.BlockSpec((1, OUT), lambda i: (0, 0)),
            pl.BlockSpec((1, OUT), lambda i: (0, 0)),
        ],
        out_specs=[
            pl.BlockSpec((NB, W2T), lambda i: (i, 0)),
            pl.BlockSpec((1, NB // 128, 128), lambda i: (i, 0, 0)),
        ],
        out_shape=[
            jax.ShapeDtypeStruct((NP, W2T), jnp.float32),
            jax.ShapeDtypeStruct((NBLK, NB // 128, 128), jnp.float32),
        ],
    )(acc1, b1.reshape(H1, HID), W2, as2.reshape(1, OUT), ad2.reshape(1, OUT))


def _ep2_body(tbl_hbm, adst_hbm, src_hbm, dst_hbm, zero_hbm, out_hbm,
              acc, src_v, dst_v, adst_v, row_a, row_b, ex_v,
              sem_a, sem_b):
    c = lax.axis_index("c")
    t = lax.axis_index("s")
    rows_per_tile = NP // NS
    w = c * NS + t
    pltpu.sync_copy(src_hbm.at[pl.ds(w * ROWS_T2, ROWS_T2)], src_v)
    pltpu.sync_copy(dst_hbm.at[pl.ds(w * ROWS_T2, ROWS_T2)], dst_v)
    pltpu.sync_copy(zero_hbm.at[pl.ds(t * rows_per_tile, rows_per_tile)],
                    acc.at[pl.ds(t * rows_per_tile, rows_per_tile)])
    pltpu.sync_copy(adst_hbm.at[0], adst_v)
    plsc.subcore_barrier()

    def compute_chunk(row_v, j):
        for g in range(C // L):
            d16 = dst_v[j, pl.ds(g * L, L)]
            av = plsc.load_gather(adst_v, [d16])
            rows = lax.iota(jnp.int32, L) + g * L
            cols = jnp.full((L,), OUT, jnp.int32)
            asv = plsc.load_gather(row_v, [rows, cols])
            al = asv + av
            al = jnp.where(al >= 0.0, al, al * jnp.float32(0.2))
            ex_v[pl.ds(g * L, L)] = jnp.exp(al)

        def scale_body(e, carry3):
            s = ex_v[pl.ds(e, L)][0]
            for r in range(W2T // L):
                row_v[e, pl.ds(r * L, L)] = row_v[e, pl.ds(r * L, L)] * s
            return carry3
        lax.fori_loop(0, C, scale_body, 0, unroll=16)

    pltpu.async_copy(tbl_hbm.at[src_v.at[0]], row_a, sem_a)

    def pair_body(jj, carry2):
        j0 = jj * 2
        j1 = j0 + 1
        pltpu.async_copy(tbl_hbm.at[src_v.at[j1]], row_b, sem_b)
        pltpu.make_async_copy(tbl_hbm.at[src_v.at[j0]], row_a, sem_a).wait()
        compute_chunk(row_a, j0)
        pltpu.sync_copy(row_a, acc.at[dst_v.at[j0]], add=True)

        @pl.when(jj < ROWS_T2 // 2 - 1)
        def _():
            pltpu.async_copy(tbl_hbm.at[src_v.at[j0 + 2]], row_a, sem_a)
        pltpu.make_async_copy(tbl_hbm.at[src_v.at[j1]], row_b, sem_b).wait()
        compute_chunk(row_b, j1)
        pltpu.sync_copy(row_b, acc.at[dst_v.at[j1]], add=True)
        return carry2
    lax.fori_loop(0, ROWS_T2 // 2, pair_body, 0)
    plsc.subcore_barrier()
    pltpu.sync_copy(acc.at[pl.ds(t * rows_per_tile, rows_per_tile)],
                    out_hbm.at[c].at[pl.ds(t * rows_per_tile, rows_per_tile)])


_ep2 = functools.partial(
    pl.kernel,
    out_type=jax.ShapeDtypeStruct((NC, NP, W2T), jnp.float32),
    mesh=plsc.VectorSubcoreMesh(core_axis_name="c", subcore_axis_name="s"),
    compiler_params=pltpu.CompilerParams(needs_layout_passes=False, use_tc_tiling_on_sc=False),
    scratch_types=[
        pltpu.VMEM_SHARED((NP, W2T), jnp.float32),
        pltpu.VMEM((ROWS_T2, C), jnp.int32),
        pltpu.VMEM((ROWS_T2, C), jnp.int32),
        pltpu.VMEM((NP,), jnp.float32),
        pltpu.VMEM((C, W2T), jnp.float32),
        pltpu.VMEM((C, W2T), jnp.float32),
        pltpu.VMEM((C + L,), jnp.float32),
        pltpu.SemaphoreType.DMA,
        pltpu.SemaphoreType.DMA,
    ],
)(_ep2_body)


def _f_body(acc_ref, b2_ref, out_ref):
    a = acc_ref[0] + acc_ref[1]
    z = a[:, 0:OUT] / (a[:, OUT + 1:OUT + 2] + 1e-16) + b2_ref[...]
    m = jnp.max(z, axis=1, keepdims=True)
    zz = z - m
    out_ref[...] = zz - jnp.log(jnp.sum(jnp.exp(zz), axis=1, keepdims=True))


def _final(acc2, b2):
    return pl.pallas_call(
        _f_body,
        grid=(NBLK,),
        in_specs=[
            pl.BlockSpec((NC, NB, W2T), lambda i: (0, i, 0)),
            pl.BlockSpec((1, OUT), lambda i: (0, 0)),
        ],
        out_specs=pl.BlockSpec((NB, OUT), lambda i: (i, 0)),
        out_shape=jax.ShapeDtypeStruct((NP, OUT), jnp.float32),
    )(acc2, b2.reshape(1, OUT))


@jax.jit
def kernel(x, edge_index, W1, att_src1, att_dst1, b1, W2, att_src2,
           att_dst2, b2):
    src = edge_index[0].astype(jnp.int32)
    dst = edge_index[1].astype(jnp.int32)
    pad = jnp.full((EP - E,), SENT, jnp.int32)
    srcf = jnp.concatenate([src, pad])
    dstf = jnp.concatenate([dst, pad])
    srcp1 = srcf.reshape(EP // C1, C1)
    dstp1 = dstf.reshape(EP // C1, C1)
    srcp = srcf.reshape(EP // C, C)
    dstp = dstf.reshape(EP // C, C)
    xp = jnp.zeros((NP, IN_DIM), jnp.float32).at[:N].set(x)
    z1 = jnp.zeros((NP, W1T), jnp.float32)
    z2 = jnp.zeros((NP, W2T), jnp.float32)

    tbl1, adst1 = _dense1(xp, W1, att_src1, att_dst1)
    acc1 = _ep1(tbl1, adst1.reshape(H1, NP), srcp1, dstp1, z1)
    tbl2, adst2 = _dense2(acc1, b1, W2, att_src2, att_dst2)
    acc2 = _ep2(tbl2, adst2.reshape(1, NP), srcp, dstp, z2)
    outp = _final(acc2, b2)
    return outp[:N]
